# baseline (speedup 1.0000x reference)
import functools

import jax
import jax.numpy as jnp
from jax import lax
from jax.experimental import pallas as pl
from jax.experimental.pallas import tpu as pltpu
from jax.experimental.pallas import tpu_sc as plsc


_NC = 2
_NS = 16
_NW = _NC * _NS
_LANES = 16
_SC_TILE = 800
_SC_NBUF = 8
_SC_FRAC = 0.25

_TC_BLOCK = 8192
_TC_NBUF = 8


@functools.lru_cache(maxsize=None)
def _make_sc_tail(rows: int, d: int, rows_sc: int):
    assert rows_sc % _NW == 0
    rows_per_w = rows_sc // _NW
    tile = _SC_TILE
    while rows_per_w % tile or tile % 8:
        tile -= 8
    steps = rows_per_w // tile
    nbuf = min(_SC_NBUF, steps)
    base0 = rows - rows_sc
    assert d % _LANES == 0 and base0 % 8 == 0

    mesh = plsc.VectorSubcoreMesh(core_axis_name="c", subcore_axis_name="s")

    @functools.partial(
        pl.kernel,
        mesh=mesh,
        out_type=jax.ShapeDtypeStruct((rows, d), jnp.float32),
        scratch_types=[
            pltpu.VMEM((tile, d), jnp.float32),
            pltpu.SemaphoreType.DMA,
        ],
    )
    def sc_tail(w_hbm, out_hbm, tile_v, sem):
        wid = lax.axis_index("s") * _NC + lax.axis_index("c")
        base = base0 + wid * rows_per_w

        pltpu.sync_copy(w_hbm, tile_v.at[pl.ds(0, 1)])
        vregs = [tile_v[0, pl.ds(_LANES * j, _LANES)] for j in range(d // _LANES)]

        def fill(r, carry):
            for j in range(d // _LANES):
                tile_v[r, pl.ds(_LANES * j, _LANES)] = vregs[j]
            return carry

        lax.fori_loop(1, tile, fill, 0)

        for t in range(nbuf):
            pltpu.async_copy(tile_v, out_hbm.at[pl.ds(base + t * tile, tile)], sem)

        def body(t, carry):
            pltpu.make_async_copy(tile_v, out_hbm.at[pl.ds(base, tile)], sem).wait()
            pltpu.async_copy(tile_v, out_hbm.at[pl.ds(base + t * tile, tile)], sem)
            return carry

        lax.fori_loop(nbuf, steps, body, 0)

        for _ in range(nbuf):
            pltpu.make_async_copy(tile_v, out_hbm.at[pl.ds(base, tile)], sem).wait()

    return sc_tail


def _make_tc_head(rows: int, d: int, rows_tc: int):
    block = _TC_BLOCK
    while rows_tc % block:
        block //= 2
    steps = rows_tc // block
    nbuf = min(_TC_NBUF, steps)

    def body(w_ref, tail_ref, o_ref, buf, sem):
        del tail_ref
        buf[...] = jnp.broadcast_to(w_ref[...], buf.shape)

        for t in range(nbuf):
            pltpu.make_async_copy(buf, o_ref.at[pl.ds(t * block, block)], sem).start()

        def ring(t, carry):
            pltpu.make_async_copy(buf, o_ref.at[pl.ds(0, block)], sem).wait()
            pltpu.make_async_copy(buf, o_ref.at[pl.ds(t * block, block)], sem).start()
            return carry

        lax.fori_loop(nbuf, steps, ring, 0)

        for _ in range(nbuf):
            pltpu.make_async_copy(buf, o_ref.at[pl.ds(0, block)], sem).wait()

    return pl.pallas_call(
        body,
        in_specs=[
            pl.BlockSpec(memory_space=pltpu.MemorySpace.VMEM),
            pl.BlockSpec(memory_space=pl.ANY),
        ],
        out_specs=pl.BlockSpec(memory_space=pl.ANY),
        out_shape=jax.ShapeDtypeStruct((rows, d), jnp.float32),
        input_output_aliases={1: 0},
        scratch_shapes=[
            pltpu.VMEM((block, d), jnp.float32),
            pltpu.SemaphoreType.DMA,
        ],
    )


def kernel(input, weight):
    B, H = input.shape
    _, D = weight.shape
    rows = B * H
    grain = _NW * _SC_TILE
    rows_sc = max(int(rows * _SC_FRAC) // grain * grain, grain)
    rows_tc = rows - rows_sc
    tail = _make_sc_tail(rows, D, rows_sc)(weight)
    out = _make_tc_head(rows, D, rows_tc)(weight, tail)
    return out.reshape(B, H, D)


# speedup vs baseline: 1.0643x; 1.0643x over previous
import functools

import jax
import jax.numpy as jnp
from jax import lax
from jax.experimental import pallas as pl
from jax.experimental.pallas import tpu as pltpu


_BLOCK_ROWS = 8192
_NBUF = 8


def _make_tc_ring(rows: int, d: int):
    block = _BLOCK_ROWS
    while rows % block:
        block //= 2
    steps = rows // block
    nbuf = min(_NBUF, steps)

    def body(w_ref, o_ref, buf, sem):
        buf[...] = jnp.broadcast_to(w_ref[...], buf.shape)

        for t in range(nbuf):
            pltpu.make_async_copy(
                buf, o_ref.at[pl.ds(t * block, block)], sem
            ).start()

        def ring(t, carry):
            pltpu.make_async_copy(buf, o_ref.at[pl.ds(0, block)], sem).wait()
            pltpu.make_async_copy(
                buf, o_ref.at[pl.ds(t * block, block)], sem
            ).start()
            return carry

        lax.fori_loop(nbuf, steps, ring, 0)

        for _ in range(nbuf):
            pltpu.make_async_copy(buf, o_ref.at[pl.ds(0, block)], sem).wait()

    return pl.pallas_call(
        body,
        in_specs=[pl.BlockSpec(memory_space=pltpu.VMEM)],
        out_specs=pl.BlockSpec(memory_space=pl.ANY),
        out_shape=jax.ShapeDtypeStruct((rows, d), jnp.float32),
        scratch_shapes=[
            pltpu.VMEM((block, d), jnp.float32),
            pltpu.SemaphoreType.DMA,
        ],
    )


def kernel(input, weight):
    B, H = input.shape
    _, D = weight.shape
    out = _make_tc_ring(B * H, D)(weight)
    return out.reshape(B, H, D)
